# Initial kernel scaffold; baseline (speedup 1.0000x reference)
import jax
import jax.numpy as jnp
from jax.experimental import pallas as pl


def kernel(input, embedding_params_diag, embedding_params_nondiag):
    raise NotImplementedError("write your pallas kernel here")


# speedup vs baseline: 1.3803x; 1.3803x over previous
import functools

import numpy as np
import jax
import jax.numpy as jnp
from jax import lax
from jax.experimental import pallas as pl
from jax.experimental.pallas import tpu as pltpu, tpu_sc as plsc

N_ANC = 80
NR = 18
NPAIR = N_ANC * (N_ANC - 1) // 2
NDIM = N_ANC * (N_ANC + 1) // 2
KPAD = 3168
NKB = KPAD // 16
BATCH = 256
NW = 32
ROWS_PER_W = BATCH // NW
ROW_ELEMS = NR * NDIM

_pi, _pj = [], []
for _iy in range(N_ANC):
    for _ix in range(_iy + 1, N_ANC):
        _pi.append(_iy)
        _pj.append(_ix)
PAIRS_I = np.array(_pi + [0] * (KPAD - NPAIR), dtype=np.int32)
PAIRS_J = np.array(_pj + [0] * (KPAD - NPAIR), dtype=np.int32)


def _sigmoid16(x):
    return 1.0 / (1.0 + jnp.exp(-x))


@functools.partial(
    pl.kernel,
    out_type=jax.ShapeDtypeStruct((BATCH, ROW_ELEMS), jnp.float32),
    mesh=plsc.VectorSubcoreMesh(core_axis_name="c", subcore_axis_name="s"),
    compiler_params=pltpu.CompilerParams(needs_layout_passes=False),
    scratch_types=[
        pltpu.VMEM((20 * N_ANC,), jnp.int32),
        pltpu.VMEM((NR * N_ANC,), jnp.int32),
        pltpu.VMEM((ROW_ELEMS,), jnp.float32),
        pltpu.VMEM((9 * KPAD,), jnp.float32),
        pltpu.VMEM((KPAD,), jnp.int32),
        pltpu.VMEM((KPAD,), jnp.int32),
        pltpu.VMEM((4 * KPAD,), jnp.float32),
        pltpu.VMEM((N_ANC,), jnp.float32),
    ],
)
def _sc_forward(inp_hbm, pd_hbm, pnd_hbm, pi_hbm, pj_hbm, out_hbm,
                inbuf, sbuf, outbuf, tab, pibuf, pjbuf, pndbuf, sigd):
    wid = lax.axis_index("s") * 2 + lax.axis_index("c")
    lanes = lax.iota(jnp.int32, 16)

    pltpu.sync_copy(pi_hbm, pibuf)
    pltpu.sync_copy(pj_hbm, pjbuf)
    pltpu.sync_copy(pnd_hbm, pndbuf)
    pltpu.sync_copy(pd_hbm, sigd)

    for ib in range(N_ANC // 16):
        sigd[pl.ds(ib * 16, 16)] = _sigmoid16(sigd[pl.ds(ib * 16, 16)])

    def table_block(kb, _):
        base = kb * 16
        f12 = _sigmoid16(pndbuf[pl.ds(0 * KPAD + base, 16)])
        f9 = _sigmoid16(pndbuf[pl.ds(1 * KPAD + base, 16)]) * f12
        f8 = _sigmoid16(pndbuf[pl.ds(2 * KPAD + base, 16)]) * f9
        f6 = _sigmoid16(pndbuf[pl.ds(3 * KPAD + base, 16)]) * f8
        tab[pl.ds(0 * KPAD + base, 16)] = jnp.zeros((16,), jnp.float32)
        tab[pl.ds(1 * KPAD + base, 16)] = f6
        tab[pl.ds(2 * KPAD + base, 16)] = f8
        tab[pl.ds(3 * KPAD + base, 16)] = f6
        tab[pl.ds(4 * KPAD + base, 16)] = f9
        tab[pl.ds(5 * KPAD + base, 16)] = f12
        tab[pl.ds(6 * KPAD + base, 16)] = f8
        tab[pl.ds(7 * KPAD + base, 16)] = f12
        tab[pl.ds(8 * KPAD + base, 16)] = jnp.ones((16,), jnp.float32)
        return 0

    lax.fori_loop(0, NKB, table_block, 0)

    def row_body(rl, _):
        b = wid * ROWS_PER_W + rl
        pltpu.sync_copy(inp_hbm.at[b], inbuf)

        for ib in range(N_ANC // 16):
            col = ib * 16

            def rec(r, carry):
                st, dl = carry
                x0 = inbuf[pl.ds(r * N_ANC + col, 16)]
                x1 = inbuf[pl.ds((r + 1) * N_ANC + col, 16)]
                x2 = inbuf[pl.ds((r + 2) * N_ANC + col, 16)]
                de = x0 + x2 - x0 * x2 * 2
                me = x1 * (1 - (x0 + x2)) + x0 * x2
                dl = dl * (1 - me * 2)
                st = jnp.clip(st + dl * de, -1, 1)
                dl = dl * (1 - st * st * (1 - me)) - st * (1 - me)
                sbuf[pl.ds(r * N_ANC + col, 16)] = st + 1
                return st, dl

            lax.fori_loop(0, NR, rec,
                          (jnp.full((16,), -1, jnp.int32),
                           jnp.full((16,), 1, jnp.int32)))

        def kb_body(kb, _):
            vi = pibuf[pl.ds(kb * 16, 16)]
            vj = pjbuf[pl.ds(kb * 16, 16)]
            tk = kb * 16 + lanes

            def r_body(r, _):
                si = plsc.load_gather(sbuf, [vi + r * N_ANC])
                sj = plsc.load_gather(sbuf, [vj + r * N_ANC])
                tv = plsc.load_gather(tab, [(si * 3 + sj) * KPAD + tk])
                outbuf[pl.ds(r * NDIM + N_ANC + kb * 16, 16)] = tv
                return 0

            lax.fori_loop(0, NR, r_body, 0)
            return 0

        lax.fori_loop(0, NKB - 1, kb_body, 0)

        vi_l = pibuf[pl.ds((NKB - 1) * 16, 16)]
        vj_l = pjbuf[pl.ds((NKB - 1) * 16, 16)]
        tk_l = (NKB - 1) * 16 + lanes
        mask_l = tk_l < NPAIR

        def r_last(r, _):
            si = plsc.load_gather(sbuf, [vi_l + r * N_ANC])
            sj = plsc.load_gather(sbuf, [vj_l + r * N_ANC])
            tv = plsc.load_gather(tab, [(si * 3 + sj) * KPAD + tk_l])
            plsc.store_scatter(outbuf, [r * NDIM + N_ANC + tk_l], tv,
                               mask=mask_l)
            return 0

        lax.fori_loop(0, NR, r_last, 0)

        def diag_body(r, _):
            for ib in range(N_ANC // 16):
                s = sbuf[pl.ds(r * N_ANC + ib * 16, 16)]
                w = sigd[pl.ds(ib * 16, 16)]
                val = (jnp.where(s == 2, 1.0, 0.0)
                       + jnp.where(s == 1, w, jnp.zeros((16,), jnp.float32)))
                outbuf[pl.ds(r * NDIM + ib * 16, 16)] = val
            return 0

        lax.fori_loop(0, NR, diag_body, 0)

        pltpu.sync_copy(outbuf, out_hbm.at[b])
        return 0

    lax.fori_loop(0, ROWS_PER_W, row_body, 0)


def kernel(input, embedding_params_diag, embedding_params_nondiag):
    inp2d = input.reshape(BATCH, 20 * N_ANC)
    pd = embedding_params_diag.reshape(N_ANC)
    pndt = jnp.pad(embedding_params_nondiag[0].T,
                   ((0, 0), (0, KPAD - NPAIR))).reshape(-1)
    out2d = _sc_forward(inp2d, pd, pndt,
                        jnp.asarray(PAIRS_I), jnp.asarray(PAIRS_J))
    return out2d.reshape(BATCH, NR, NDIM)


# speedup vs baseline: 2.7864x; 2.0186x over previous
import functools

import numpy as np
import jax
import jax.numpy as jnp
from jax import lax
from jax.experimental import pallas as pl
from jax.experimental.pallas import tpu as pltpu, tpu_sc as plsc

N_ANC = 80
NR = 18
NRH = NR // 2
NPAIR = N_ANC * (N_ANC - 1) // 2
NDIM = N_ANC * (N_ANC + 1) // 2
KPAD = 3168
NKB = KPAD // 16
BATCH = 256
NW = 32
ROWS_PER_W = BATCH // NW
HALF_ELEMS = NRH * NDIM

_pi, _pj = [], []
for _iy in range(N_ANC):
    for _ix in range(_iy + 1, N_ANC):
        _pi.append(_iy)
        _pj.append(_ix)
PAIRS_I = np.array(_pi + [0] * (KPAD - NPAIR), dtype=np.int32)
PAIRS_J = np.array(_pj + [0] * (KPAD - NPAIR), dtype=np.int32)


def _sigmoid16(x):
    return 1.0 / (1.0 + jnp.exp(-x))


@functools.partial(
    pl.kernel,
    out_type=jax.ShapeDtypeStruct((BATCH * 2, HALF_ELEMS), jnp.float32),
    mesh=plsc.VectorSubcoreMesh(core_axis_name="c", subcore_axis_name="s"),
    compiler_params=pltpu.CompilerParams(needs_layout_passes=False),
    scratch_types=[
        pltpu.VMEM((20 * N_ANC,), jnp.int32),
        pltpu.VMEM((NR * N_ANC,), jnp.int32),
        pltpu.VMEM((HALF_ELEMS,), jnp.float32),
        pltpu.VMEM((HALF_ELEMS,), jnp.float32),
        pltpu.VMEM((9 * KPAD,), jnp.float32),
        pltpu.VMEM((KPAD,), jnp.int32),
        pltpu.VMEM((KPAD,), jnp.int32),
        pltpu.VMEM((4 * KPAD,), jnp.float32),
        pltpu.VMEM((N_ANC,), jnp.float32),
        pltpu.SemaphoreType.DMA,
        pltpu.SemaphoreType.DMA,
    ],
)
def _sc_forward(inp_hbm, pd_hbm, pnd_hbm, pi_hbm, pj_hbm, out_hbm,
                inbuf, sbuf, outbuf0, outbuf1, tab, pibuf, pjbuf, pndbuf, sigd,
                sem0, sem1):
    outbufs = (outbuf0, outbuf1)
    wid = lax.axis_index("s") * 2 + lax.axis_index("c")
    lanes = lax.iota(jnp.int32, 16)
    sems = (sem0, sem1)

    pltpu.sync_copy(pi_hbm, pibuf)
    pltpu.sync_copy(pj_hbm, pjbuf)
    pltpu.sync_copy(pnd_hbm, pndbuf)
    pltpu.sync_copy(pd_hbm, sigd)

    for ib in range(N_ANC // 16):
        sigd[pl.ds(ib * 16, 16)] = _sigmoid16(sigd[pl.ds(ib * 16, 16)])

    def table_block(kb, _):
        base = kb * 16
        f12 = _sigmoid16(pndbuf[pl.ds(0 * KPAD + base, 16)])
        f9 = _sigmoid16(pndbuf[pl.ds(1 * KPAD + base, 16)]) * f12
        f8 = _sigmoid16(pndbuf[pl.ds(2 * KPAD + base, 16)]) * f9
        f6 = _sigmoid16(pndbuf[pl.ds(3 * KPAD + base, 16)]) * f8
        tab[pl.ds(0 * KPAD + base, 16)] = jnp.zeros((16,), jnp.float32)
        tab[pl.ds(1 * KPAD + base, 16)] = f6
        tab[pl.ds(2 * KPAD + base, 16)] = f8
        tab[pl.ds(3 * KPAD + base, 16)] = f6
        tab[pl.ds(4 * KPAD + base, 16)] = f9
        tab[pl.ds(5 * KPAD + base, 16)] = f12
        tab[pl.ds(6 * KPAD + base, 16)] = f8
        tab[pl.ds(7 * KPAD + base, 16)] = f12
        tab[pl.ds(8 * KPAD + base, 16)] = jnp.ones((16,), jnp.float32)
        return 0

    lax.fori_loop(0, NKB, table_block, 0)

    def row_body(rl, _):
        b = wid * ROWS_PER_W + rl
        pltpu.sync_copy(inp_hbm.at[b], inbuf)

        for ib in range(N_ANC // 16):
            col = ib * 16

            def rec(r, carry):
                st, dl = carry
                x0 = inbuf[pl.ds(r * N_ANC + col, 16)]
                x1 = inbuf[pl.ds((r + 1) * N_ANC + col, 16)]
                x2 = inbuf[pl.ds((r + 2) * N_ANC + col, 16)]
                de = x0 + x2 - x0 * x2 * 2
                me = x1 * (1 - (x0 + x2)) + x0 * x2
                dl = dl * (1 - me * 2)
                st = jnp.clip(st + dl * de, -1, 1)
                dl = dl * (1 - st * st * (1 - me)) - st * (1 - me)
                sbuf[pl.ds(r * N_ANC + col, 16)] = st + 1
                return st, dl

            lax.fori_loop(0, NR, rec,
                          (jnp.full((16,), -1, jnp.int32),
                           jnp.full((16,), 1, jnp.int32)))

        for h in range(2):
            outbuf = outbufs[h]
            @plsc.parallel_loop(0, NKB - 1)
            def kb_body(kb):
                vi = pibuf[pl.ds(kb * 16, 16)]
                vj = pjbuf[pl.ds(kb * 16, 16)]
                tk = kb * 16 + lanes
                tvs = []
                for r in range(NRH):
                    rr = h * NRH + r
                    si = plsc.load_gather(sbuf, [vi + rr * N_ANC])
                    sj = plsc.load_gather(sbuf, [vj + rr * N_ANC])
                    tvs.append(
                        plsc.load_gather(tab, [(si * 3 + sj) * KPAD + tk]))
                for r, tv in enumerate(tvs):
                    outbuf[pl.ds(r * NDIM + N_ANC + kb * 16, 16)] = tv

            vi_l = pibuf[pl.ds((NKB - 1) * 16, 16)]
            vj_l = pjbuf[pl.ds((NKB - 1) * 16, 16)]
            tk_l = (NKB - 1) * 16 + lanes
            mask_l = tk_l < NPAIR
            for r in range(NRH):
                rr = h * NRH + r
                si = plsc.load_gather(sbuf, [vi_l + rr * N_ANC])
                sj = plsc.load_gather(sbuf, [vj_l + rr * N_ANC])
                tv = plsc.load_gather(tab, [(si * 3 + sj) * KPAD + tk_l])
                plsc.store_scatter(outbuf, [r * NDIM + N_ANC + tk_l],
                                   tv, mask=mask_l)

            for r in range(NRH):
                rr = h * NRH + r
                for ib in range(N_ANC // 16):
                    s = sbuf[pl.ds(rr * N_ANC + ib * 16, 16)]
                    w = sigd[pl.ds(ib * 16, 16)]
                    val = (jnp.where(s == 2, 1.0, 0.0)
                           + jnp.where(s == 1, w, jnp.zeros((16,), jnp.float32)))
                    outbuf[pl.ds(r * NDIM + ib * 16, 16)] = val

            pltpu.sync_copy(outbuf, out_hbm.at[2 * b + h])
        return 0

    lax.fori_loop(0, ROWS_PER_W, row_body, 0)


def kernel(input, embedding_params_diag, embedding_params_nondiag):
    inp2d = input.reshape(BATCH, 20 * N_ANC)
    pd = embedding_params_diag.reshape(N_ANC)
    pndt = jnp.pad(embedding_params_nondiag[0].T,
                   ((0, 0), (0, KPAD - NPAIR))).reshape(-1)
    out2d = _sc_forward(inp2d, pd, pndt,
                        jnp.asarray(PAIRS_I), jnp.asarray(PAIRS_J))
    return out2d.reshape(BATCH, NR, NDIM)
